# baseline (speedup 1.0000x reference)
import jax
import jax.numpy as jnp
from jax.experimental import pallas as pl
from jax.experimental.pallas import tpu as pltpu

BETA = 0.25
D = 64
K = 1024
HW = 1024
BPG = 4


def _vq_body(lat_ref, cb_ref, out_ref, loss_ref):
    x = lat_ref[...]
    c = cb_ref[...]
    c2 = c + c
    b = jnp.sum(c * c, axis=1, keepdims=True)
    part = jnp.float32(0.0)
    for i in range(BPG):
        xb = x[i]
        a = jnp.sum(xb * xb, axis=0, keepdims=True)
        mm2 = jax.lax.dot_general(
            c2, xb, (((1,), (0,)), ((), ())),
            preferred_element_type=jnp.float32)
        dist = (a + b) - mm2
        m = jnp.min(dist, axis=0, keepdims=True)
        iota = jax.lax.broadcasted_iota(jnp.int32, (K, HW), 0)
        idx = jnp.min(jnp.where(dist == m, iota, jnp.int32(K)), axis=0,
                      keepdims=True)
        onehot = (iota == idx).astype(jnp.float32)
        q = jax.lax.dot_general(
            c, onehot, (((0,), (0,)), ((), ())),
            preferred_element_type=jnp.float32)
        out_ref[i] = q
        part = part + jnp.sum(m)
    prev = jnp.where(pl.program_id(0) == 0, 0.0, loss_ref[0, 0])
    loss_ref[0, 0] = prev + part


def kernel(latents, codebook):
    B, d, H, W = latents.shape
    lat3 = latents.reshape(B, d, H * W)
    n = B * H * W
    grid = B // BPG
    q3, loss = pl.pallas_call(
        _vq_body,
        grid=(grid,),
        in_specs=[
            pl.BlockSpec((BPG, d, HW), lambda i: (i, 0, 0)),
            pl.BlockSpec((K, d), lambda i: (0, 0)),
        ],
        out_specs=[
            pl.BlockSpec((BPG, d, HW), lambda i: (i, 0, 0)),
            pl.BlockSpec(memory_space=pltpu.SMEM, block_shape=(1, 1),
                         index_map=lambda i: (0, 0)),
        ],
        out_shape=[
            jax.ShapeDtypeStruct((B, d, HW), jnp.float32),
            jax.ShapeDtypeStruct((1, 1), jnp.float32),
        ],
    )(lat3, codebook)
    quantized = q3.reshape(B, d, H, W)
    vq_loss = (1.0 + BETA) * loss[0, 0] / (n * d)
    return quantized, vq_loss
